# Initial kernel scaffold; baseline (speedup 1.0000x reference)
import jax
import jax.numpy as jnp
from jax.experimental import pallas as pl


def kernel(x_local, x_global, edge_index, batch, W_local, b_local, W_global, b_global, W_mix, b_mix, W_msg, b_msg, W_self, b_self, W_out, b_out):
    raise NotImplementedError("write your pallas kernel here")


# speedup vs baseline: 6.2793x; 6.2793x over previous
import functools

import jax
import jax.numpy as jnp
from jax import lax
from jax.experimental import pallas as pl
from jax.experimental.pallas import tpu as pltpu
from jax.experimental.pallas import tpu_sc as plsc

N = 100000
E = 1600000
B = 128
HIDDEN = 32
HALF = 16

ROWS = 1000
GRID = N // ROWS

NTILES = 16
CHUNK = 80
CHUNK_ROWS = E // CHUNK
STAGE = 40
NSUPER = CHUNK_ROWS // STAGE
NSUPER_PER_TILE = -(-NSUPER // NTILES)
INIT_CHUNK = 1000
NINIT = N // INIT_CHUNK
NINIT_PER_TILE = -(-NINIT // NTILES)


def _encoder_body(x_ref, b_ref, xg_ref, Wl_ref, bl_ref, Wg_ref, bg_ref,
                  Wm_ref, bm_ref, Wmsg_ref, bmsg_ref, Ws_ref, bs_ref,
                  m0a_ref, m0b_ref, t0a_ref, t0b_ref):
    hg_tab = jnp.maximum(
        jnp.dot(xg_ref[...], Wg_ref[...], preferred_element_type=jnp.float32)
        + bg_ref[...], 0.0)
    hl = jnp.maximum(
        jnp.dot(x_ref[...], Wl_ref[...], preferred_element_type=jnp.float32)
        + bl_ref[...], 0.0)
    b = b_ref[0, 0, :]
    onehot = (b[:, None] == lax.broadcasted_iota(jnp.int32, (ROWS, B), 1)
              ).astype(jnp.float32)
    hgl = jnp.dot(onehot, hg_tab, preferred_element_type=jnp.float32)
    hi = hl * hgl
    Wm = Wm_ref[...]
    h0 = jnp.maximum(
        jnp.dot(hl, Wm[0:32, :], preferred_element_type=jnp.float32)
        + jnp.dot(hgl, Wm[32:64, :], preferred_element_type=jnp.float32)
        + jnp.dot(hi, Wm[64:96, :], preferred_element_type=jnp.float32)
        + bm_ref[...], 0.0)
    m0 = jnp.maximum(
        jnp.dot(h0, Wmsg_ref[...], preferred_element_type=jnp.float32)
        + bmsg_ref[...], 0.0)
    t0 = m0 + jnp.dot(h0, Ws_ref[...], preferred_element_type=jnp.float32) \
        + bs_ref[...]
    m0a_ref[...] = m0[:, 0:HALF]
    m0b_ref[...] = m0[:, HALF:HIDDEN]
    t0a_ref[...] = t0[:, 0:HALF]
    t0b_ref[...] = t0[:, HALF:HIDDEN]


def _encoder(x_local, batch3, x_global, W_local, b_local, W_global, b_global,
             W_mix, b_mix, W_msg, b_msg, W_self, b_self):
    full = lambda shape: pl.BlockSpec(shape, lambda i: (0,) * len(shape))
    out = jax.ShapeDtypeStruct((N, HALF), jnp.float32)
    return pl.pallas_call(
        _encoder_body,
        grid=(GRID,),
        in_specs=[
            pl.BlockSpec((ROWS, 128), lambda i: (i, 0)),
            pl.BlockSpec((1, 1, ROWS), lambda i: (i, 0, 0)),
            full((B, 64)), full((128, HIDDEN)), full((1, HIDDEN)),
            full((64, HIDDEN)), full((1, HIDDEN)),
            full((96, HIDDEN)), full((1, HIDDEN)),
            full((HIDDEN, HIDDEN)), full((1, HIDDEN)),
            full((HIDDEN, HIDDEN)), full((1, HIDDEN)),
        ],
        out_specs=[pl.BlockSpec((ROWS, HALF), lambda i: (i, 0))] * 4,
        out_shape=[out, out, out, out],
    )(x_local, batch3, x_global, W_local, b_local, W_global, b_global,
      W_mix, b_mix, W_msg, b_msg, W_self, b_self)


def _sc_scatter(edges3, m0a, m0b, t0a, t0b):
    mesh = plsc.VectorSubcoreMesh(core_axis_name="c", subcore_axis_name="s")

    @functools.partial(
        pl.kernel, mesh=mesh,
        compiler_params=pltpu.CompilerParams(use_tc_tiling_on_sc=False),
        out_type=[jax.ShapeDtypeStruct((N, HALF), jnp.float32),
                  jax.ShapeDtypeStruct((N, HALF), jnp.float32)],
        scratch_types=[
            pltpu.VMEM((STAGE, CHUNK), jnp.int32),
            pltpu.VMEM((STAGE, CHUNK), jnp.int32),
            pltpu.VMEM((CHUNK, HALF), jnp.float32),
            pltpu.VMEM((INIT_CHUNK, HALF), jnp.float32),

            pltpu.VMEM_SHARED((N, HALF), jnp.float32),
            pltpu.SemaphoreType.DMA,
        ],
    )
    def sc_fn(e3, m0a_h, m0b_h, t0a_h, t0b_h, outa_h, outb_h,
              src_v, dst_v, rows_v, bounce_v, acc_sh, sem):
        c = lax.axis_index("c")
        s = lax.axis_index("s")

        def run(table_h, t0_h, out_h):
            def init_k(k, _):
                idx = s + k * NTILES

                @pl.when(idx < NINIT)
                def _():
                    r0 = pl.multiple_of(idx * INIT_CHUNK, 8)
                    pltpu.sync_copy(t0_h.at[pl.ds(r0, INIT_CHUNK), :],
                                    bounce_v)
                    pltpu.sync_copy(bounce_v,
                                    acc_sh.at[pl.ds(r0, INIT_CHUNK), :])
                return _
            lax.fori_loop(0, NINIT_PER_TILE, init_k, None)
            plsc.subcore_barrier()

            def super_k(g, _):
                idx = s + g * NTILES

                @pl.when(idx < NSUPER)
                def _():
                    row0 = pl.multiple_of(idx * STAGE, 8)
                    pltpu.sync_copy(e3.at[0, pl.ds(row0, STAGE), :], src_v)
                    pltpu.sync_copy(e3.at[1, pl.ds(row0, STAGE), :], dst_v)

                    def chunk_k(j, _):
                        pltpu.async_copy(table_h.at[src_v.at[j]], rows_v,
                                         sem).wait()
                        pltpu.sync_copy(rows_v, acc_sh.at[dst_v.at[j]],
                                        add=True)
                        return _
                    lax.fori_loop(0, STAGE, chunk_k, None)
                return _
            lax.fori_loop(0, NSUPER_PER_TILE, super_k, None)
            plsc.subcore_barrier()

            def wb_k(k, _):
                idx = s + k * NTILES

                @pl.when(idx < NINIT)
                def _():
                    r0 = pl.multiple_of(idx * INIT_CHUNK, 8)
                    pltpu.sync_copy(acc_sh.at[pl.ds(r0, INIT_CHUNK), :],
                                    bounce_v)
                    pltpu.sync_copy(bounce_v,
                                    out_h.at[pl.ds(r0, INIT_CHUNK), :])
                return _
            lax.fori_loop(0, NINIT_PER_TILE, wb_k, None)

        @pl.when(c == 0)
        def _():
            run(m0a_h, t0a_h, outa_h)

        @pl.when(c == 1)
        def _():
            run(m0b_h, t0b_h, outb_h)

    return sc_fn(edges3, m0a, m0b, t0a, t0b)


def _head_body(aa_ref, ab_ref, Wo_ref, bo_ref, out_ref):
    h = jnp.maximum(jnp.concatenate([aa_ref[...], ab_ref[...]], axis=1), 0.0)
    out_ref[...] = jnp.dot(h, Wo_ref[...], preferred_element_type=jnp.float32) \
        + bo_ref[...]


def _head(acca, accb, W_out, b_out):
    full = lambda shape: pl.BlockSpec(shape, lambda i: (0,) * len(shape))
    return pl.pallas_call(
        _head_body,
        grid=(GRID,),
        in_specs=[
            pl.BlockSpec((ROWS, HALF), lambda i: (i, 0)),
            pl.BlockSpec((ROWS, HALF), lambda i: (i, 0)),
            full((HIDDEN, 2)), full((1, 2)),
        ],
        out_specs=pl.BlockSpec((ROWS, 2), lambda i: (i, 0)),
        out_shape=jax.ShapeDtypeStruct((N, 2), jnp.float32),
    )(acca, accb, W_out, b_out)


def kernel(x_local, x_global, edge_index, batch, W_local, b_local, W_global,
           b_global, W_mix, b_mix, W_msg, b_msg, W_self, b_self, W_out, b_out):
    batch3 = batch.reshape(GRID, 1, ROWS)
    edges3 = edge_index.reshape(2, CHUNK_ROWS, CHUNK)
    m0a, m0b, t0a, t0b = _encoder(
        x_local, batch3, x_global, W_local, b_local.reshape(1, HIDDEN),
        W_global, b_global.reshape(1, HIDDEN), W_mix, b_mix.reshape(1, HIDDEN),
        W_msg, b_msg.reshape(1, HIDDEN), W_self, b_self.reshape(1, HIDDEN))
    acca, accb = _sc_scatter(edges3, m0a, m0b, t0a, t0b)
    return _head(acca, accb, W_out, b_out.reshape(1, 2))


# speedup vs baseline: 9.0674x; 1.4440x over previous
import functools

import jax
import jax.numpy as jnp
from jax import lax
from jax.experimental import pallas as pl
from jax.experimental.pallas import tpu as pltpu
from jax.experimental.pallas import tpu_sc as plsc

N = 100000
E = 1600000
B = 128
HIDDEN = 32
HALF = 16

ROWS = 1000
GRID = N // ROWS

NTILES = 16
CHUNK = 80
CHUNK_ROWS = E // CHUNK
STAGE = 40
NSUPER = CHUNK_ROWS // STAGE
NSUPER_PER_TILE = -(-NSUPER // NTILES)
INIT_CHUNK = 1000
NINIT = N // INIT_CHUNK
NINIT_PER_TILE = -(-NINIT // NTILES)


def _encoder_body(x_ref, b_ref, xg_ref, Wl_ref, bl_ref, Wg_ref, bg_ref,
                  Wm_ref, bm_ref, Wmsg_ref, bmsg_ref, Ws_ref, bs_ref,
                  m0a_ref, m0b_ref, t0a_ref, t0b_ref):
    hg_tab = jnp.maximum(
        jnp.dot(xg_ref[...], Wg_ref[...], preferred_element_type=jnp.float32)
        + bg_ref[...], 0.0)
    hl = jnp.maximum(
        jnp.dot(x_ref[...], Wl_ref[...], preferred_element_type=jnp.float32)
        + bl_ref[...], 0.0)
    b = b_ref[0, 0, :]
    onehot = (b[:, None] == lax.broadcasted_iota(jnp.int32, (ROWS, B), 1)
              ).astype(jnp.float32)
    hgl = jnp.dot(onehot, hg_tab, preferred_element_type=jnp.float32)
    hi = hl * hgl
    Wm = Wm_ref[...]
    h0 = jnp.maximum(
        jnp.dot(hl, Wm[0:32, :], preferred_element_type=jnp.float32)
        + jnp.dot(hgl, Wm[32:64, :], preferred_element_type=jnp.float32)
        + jnp.dot(hi, Wm[64:96, :], preferred_element_type=jnp.float32)
        + bm_ref[...], 0.0)
    m0 = jnp.maximum(
        jnp.dot(h0, Wmsg_ref[...], preferred_element_type=jnp.float32)
        + bmsg_ref[...], 0.0)
    t0 = m0 + jnp.dot(h0, Ws_ref[...], preferred_element_type=jnp.float32) \
        + bs_ref[...]
    m0a_ref[...] = m0[:, 0:HALF]
    m0b_ref[...] = m0[:, HALF:HIDDEN]
    t0a_ref[...] = t0[:, 0:HALF]
    t0b_ref[...] = t0[:, HALF:HIDDEN]


def _encoder(x_local, batch3, x_global, W_local, b_local, W_global, b_global,
             W_mix, b_mix, W_msg, b_msg, W_self, b_self):
    full = lambda shape: pl.BlockSpec(shape, lambda i: (0,) * len(shape))
    out = jax.ShapeDtypeStruct((N, HALF), jnp.float32)
    return pl.pallas_call(
        _encoder_body,
        grid=(GRID,),
        in_specs=[
            pl.BlockSpec((ROWS, 128), lambda i: (i, 0)),
            pl.BlockSpec((1, 1, ROWS), lambda i: (i, 0, 0)),
            full((B, 64)), full((128, HIDDEN)), full((1, HIDDEN)),
            full((64, HIDDEN)), full((1, HIDDEN)),
            full((96, HIDDEN)), full((1, HIDDEN)),
            full((HIDDEN, HIDDEN)), full((1, HIDDEN)),
            full((HIDDEN, HIDDEN)), full((1, HIDDEN)),
        ],
        out_specs=[pl.BlockSpec((ROWS, HALF), lambda i: (i, 0))] * 4,
        out_shape=[out, out, out, out],
    )(x_local, batch3, x_global, W_local, b_local, W_global, b_global,
      W_mix, b_mix, W_msg, b_msg, W_self, b_self)


def _sc_scatter(edges3, m0a, m0b, t0a, t0b):
    mesh = plsc.VectorSubcoreMesh(core_axis_name="c", subcore_axis_name="s")

    @functools.partial(
        pl.kernel, mesh=mesh,
        compiler_params=pltpu.CompilerParams(use_tc_tiling_on_sc=False),
        out_type=[jax.ShapeDtypeStruct((N, HALF), jnp.float32),
                  jax.ShapeDtypeStruct((N, HALF), jnp.float32)],
        scratch_types=[
            pltpu.VMEM((STAGE, CHUNK), jnp.int32),
            pltpu.VMEM((STAGE, CHUNK), jnp.int32),
            pltpu.VMEM((CHUNK, HALF), jnp.float32),
            pltpu.VMEM((CHUNK, HALF), jnp.float32),
            pltpu.VMEM((INIT_CHUNK, HALF), jnp.float32),
            pltpu.VMEM_SHARED((N, HALF), jnp.float32),
            pltpu.SemaphoreType.DMA,
            pltpu.SemaphoreType.DMA,
        ],
    )
    def sc_fn(e3, m0a_h, m0b_h, t0a_h, t0b_h, outa_h, outb_h,
              src_v, dst_v, rows0_v, rows1_v, bounce_v, acc_sh, sem0, sem1):
        c = lax.axis_index("c")
        s = lax.axis_index("s")

        def run(table_h, t0_h, out_h):
            def init_k(k, _):
                idx = s + k * NTILES

                @pl.when(idx < NINIT)
                def _():
                    r0 = pl.multiple_of(idx * INIT_CHUNK, 8)
                    pltpu.sync_copy(t0_h.at[pl.ds(r0, INIT_CHUNK), :],
                                    bounce_v)
                    pltpu.sync_copy(bounce_v,
                                    acc_sh.at[pl.ds(r0, INIT_CHUNK), :])
                return _
            lax.fori_loop(0, NINIT_PER_TILE, init_k, None)
            plsc.subcore_barrier()

            def super_k(g, _):
                idx = s + g * NTILES

                @pl.when(idx < NSUPER)
                def _():
                    row0 = pl.multiple_of(idx * STAGE, 8)
                    pltpu.sync_copy(e3.at[0, pl.ds(row0, STAGE), :], src_v)
                    pltpu.sync_copy(e3.at[1, pl.ds(row0, STAGE), :], dst_v)

                    pltpu.async_copy(table_h.at[src_v.at[0]], rows0_v, sem0)

                    def pair_k(j, _):
                        c0 = 2 * j
                        pltpu.async_copy(table_h.at[src_v.at[c0 + 1]],
                                         rows1_v, sem1)
                        pltpu.make_async_copy(table_h.at[src_v.at[c0]],
                                              rows0_v, sem0).wait()
                        pltpu.sync_copy(rows0_v, acc_sh.at[dst_v.at[c0]],
                                        add=True)

                        @pl.when(j < STAGE // 2 - 1)
                        def _():
                            pltpu.async_copy(table_h.at[src_v.at[c0 + 2]],
                                             rows0_v, sem0)
                        pltpu.make_async_copy(table_h.at[src_v.at[c0 + 1]],
                                              rows1_v, sem1).wait()
                        pltpu.sync_copy(rows1_v, acc_sh.at[dst_v.at[c0 + 1]],
                                        add=True)
                        return _
                    lax.fori_loop(0, STAGE // 2, pair_k, None)
                return _
            lax.fori_loop(0, NSUPER_PER_TILE, super_k, None)
            plsc.subcore_barrier()

            def wb_k(k, _):
                idx = s + k * NTILES

                @pl.when(idx < NINIT)
                def _():
                    r0 = pl.multiple_of(idx * INIT_CHUNK, 8)
                    pltpu.sync_copy(acc_sh.at[pl.ds(r0, INIT_CHUNK), :],
                                    bounce_v)
                    pltpu.sync_copy(bounce_v,
                                    out_h.at[pl.ds(r0, INIT_CHUNK), :])
                return _
            lax.fori_loop(0, NINIT_PER_TILE, wb_k, None)

        @pl.when(c == 0)
        def _():
            run(m0a_h, t0a_h, outa_h)

        @pl.when(c == 1)
        def _():
            run(m0b_h, t0b_h, outb_h)

    return sc_fn(edges3, m0a, m0b, t0a, t0b)


def _head_body(aa_ref, ab_ref, Wo_ref, bo_ref, out_ref):
    h = jnp.maximum(jnp.concatenate([aa_ref[...], ab_ref[...]], axis=1), 0.0)
    out_ref[...] = jnp.dot(h, Wo_ref[...], preferred_element_type=jnp.float32) \
        + bo_ref[...]


def _head(acca, accb, W_out, b_out):
    full = lambda shape: pl.BlockSpec(shape, lambda i: (0,) * len(shape))
    return pl.pallas_call(
        _head_body,
        grid=(GRID,),
        in_specs=[
            pl.BlockSpec((ROWS, HALF), lambda i: (i, 0)),
            pl.BlockSpec((ROWS, HALF), lambda i: (i, 0)),
            full((HIDDEN, 2)), full((1, 2)),
        ],
        out_specs=pl.BlockSpec((ROWS, 2), lambda i: (i, 0)),
        out_shape=jax.ShapeDtypeStruct((N, 2), jnp.float32),
    )(acca, accb, W_out, b_out)


def kernel(x_local, x_global, edge_index, batch, W_local, b_local, W_global,
           b_global, W_mix, b_mix, W_msg, b_msg, W_self, b_self, W_out, b_out):
    batch3 = batch.reshape(GRID, 1, ROWS)
    edges3 = edge_index.reshape(2, CHUNK_ROWS, CHUNK)
    m0a, m0b, t0a, t0b = _encoder(
        x_local, batch3, x_global, W_local, b_local.reshape(1, HIDDEN),
        W_global, b_global.reshape(1, HIDDEN), W_mix, b_mix.reshape(1, HIDDEN),
        W_msg, b_msg.reshape(1, HIDDEN), W_self, b_self.reshape(1, HIDDEN))
    acca, accb = _sc_scatter(edges3, m0a, m0b, t0a, t0b)
    return _head(acca, accb, W_out, b_out.reshape(1, 2))


# speedup vs baseline: 9.7856x; 1.0792x over previous
import functools

import jax
import jax.numpy as jnp
from jax import lax
from jax.experimental import pallas as pl
from jax.experimental.pallas import tpu as pltpu
from jax.experimental.pallas import tpu_sc as plsc

N = 100000
E = 1600000
B = 128
HIDDEN = 32
HALF = 16

ROWS = 1024
GRID = -(-N // ROWS)
PACK = ROWS // 8
NPAD = GRID * ROWS
N8P = GRID * PACK


NTILES = 16
CHUNK = 80
CHUNK_ROWS = E // CHUNK
STAGE = 40
NSUPER = CHUNK_ROWS // STAGE
NSUPER_PER_TILE = -(-NSUPER // NTILES)
INIT_CHUNK = 1024
NINIT = NPAD // INIT_CHUNK
NINIT_PER_TILE = -(-NINIT // NTILES)


def _encoder_body(x_ref, b_ref, xg_ref, Wl_ref, bl_ref, Wg_ref, bg_ref,
                  Wm_ref, bm_ref, Wmsg_ref, bmsg_ref, Ws_ref, bs_ref,
                  m0a_ref, m0b_ref, t0a_ref, t0b_ref):
    hg_tab = jnp.maximum(
        jnp.dot(xg_ref[...], Wg_ref[...], preferred_element_type=jnp.float32)
        + bg_ref[...], 0.0)
    Wm = Wm_ref[...]
    for k in range(8):
        r0 = PACK * k
        hl = jnp.maximum(
            jnp.dot(x_ref[r0:r0 + PACK, :], Wl_ref[...],
                    preferred_element_type=jnp.float32) + bl_ref[...], 0.0)
        b = b_ref[0, r0:r0 + PACK]
        onehot = (b[:, None] == lax.broadcasted_iota(jnp.int32, (PACK, B), 1)
                  ).astype(jnp.float32)
        hgl = jnp.dot(onehot, hg_tab, preferred_element_type=jnp.float32)
        hi = hl * hgl
        h0 = jnp.maximum(
            jnp.dot(hl, Wm[0:32, :], preferred_element_type=jnp.float32)
            + jnp.dot(hgl, Wm[32:64, :], preferred_element_type=jnp.float32)
            + jnp.dot(hi, Wm[64:96, :], preferred_element_type=jnp.float32)
            + bm_ref[...], 0.0)
        m0 = jnp.maximum(
            jnp.dot(h0, Wmsg_ref[...], preferred_element_type=jnp.float32)
            + bmsg_ref[...], 0.0)
        t0 = m0 + jnp.dot(h0, Ws_ref[...],
                          preferred_element_type=jnp.float32) + bs_ref[...]
        nid0 = pl.program_id(0) * ROWS + r0
        valid = (nid0 + lax.broadcasted_iota(jnp.int32, (PACK, 1), 0)) < N
        m0 = jnp.where(valid, m0, 0.0)
        t0 = jnp.where(valid, t0, 0.0)
        m0a_ref[:, HALF * k:HALF * (k + 1)] = m0[:, 0:HALF]
        m0b_ref[:, HALF * k:HALF * (k + 1)] = m0[:, HALF:HIDDEN]
        t0a_ref[:, HALF * k:HALF * (k + 1)] = t0[:, 0:HALF]
        t0b_ref[:, HALF * k:HALF * (k + 1)] = t0[:, HALF:HIDDEN]


def _encoder(x_local, batch3, x_global, W_local, b_local, W_global, b_global,
             W_mix, b_mix, W_msg, b_msg, W_self, b_self):
    full = lambda shape: pl.BlockSpec(shape, lambda i: (0,) * len(shape))
    out = jax.ShapeDtypeStruct((N8P, 128), jnp.float32)
    return pl.pallas_call(
        _encoder_body,
        grid=(GRID,),
        in_specs=[
            pl.BlockSpec((ROWS, 128), lambda i: (i, 0)),
            pl.BlockSpec((1, ROWS), lambda i: (0, i)),
            full((B, 64)), full((128, HIDDEN)), full((1, HIDDEN)),
            full((64, HIDDEN)), full((1, HIDDEN)),
            full((96, HIDDEN)), full((1, HIDDEN)),
            full((HIDDEN, HIDDEN)), full((1, HIDDEN)),
            full((HIDDEN, HIDDEN)), full((1, HIDDEN)),
        ],
        out_specs=[pl.BlockSpec((PACK, 128), lambda i: (i, 0))] * 4,
        out_shape=[out, out, out, out],
    )(x_local, batch3, x_global, W_local, b_local, W_global, b_global,
      W_mix, b_mix, W_msg, b_msg, W_self, b_self)


def _sc_scatter(edges3, m0a, m0b, t0a, t0b):
    mesh = plsc.VectorSubcoreMesh(core_axis_name="c", subcore_axis_name="s")

    @functools.partial(
        pl.kernel, mesh=mesh,
        compiler_params=pltpu.CompilerParams(use_tc_tiling_on_sc=False),
        out_type=[jax.ShapeDtypeStruct((NPAD, HALF), jnp.float32),
                  jax.ShapeDtypeStruct((NPAD, HALF), jnp.float32)],
        scratch_types=[
            pltpu.VMEM((STAGE, CHUNK), jnp.int32),
            pltpu.VMEM((STAGE, CHUNK), jnp.int32),
            pltpu.VMEM((CHUNK, HALF), jnp.float32),
            pltpu.VMEM((CHUNK, HALF), jnp.float32),
            pltpu.VMEM((INIT_CHUNK, HALF), jnp.float32),
            pltpu.VMEM_SHARED((NPAD, HALF), jnp.float32),
            pltpu.SemaphoreType.DMA,
            pltpu.SemaphoreType.DMA,
        ],
    )
    def sc_fn(e3, m0a_h, m0b_h, t0a_h, t0b_h, outa_h, outb_h,
              src_v, dst_v, rows0_v, rows1_v, bounce_v, acc_sh, sem0, sem1):
        c = lax.axis_index("c")
        s = lax.axis_index("s")

        def run(table_h, t0_h, out_h):
            def init_k(k, _):
                idx = s + k * NTILES

                @pl.when(idx < NINIT)
                def _():
                    r0 = pl.multiple_of(idx * INIT_CHUNK, 8)
                    pltpu.sync_copy(t0_h.at[pl.ds(r0, INIT_CHUNK), :],
                                    bounce_v)
                    pltpu.sync_copy(bounce_v,
                                    acc_sh.at[pl.ds(r0, INIT_CHUNK), :])
                return _
            lax.fori_loop(0, NINIT_PER_TILE, init_k, None)
            plsc.subcore_barrier()

            def super_k(g, _):
                idx = s + g * NTILES

                @pl.when(idx < NSUPER)
                def _():
                    row0 = pl.multiple_of(idx * STAGE, 8)
                    pltpu.sync_copy(e3.at[0, pl.ds(row0, STAGE), :], src_v)
                    pltpu.sync_copy(e3.at[1, pl.ds(row0, STAGE), :], dst_v)

                    pltpu.async_copy(table_h.at[src_v.at[0]], rows0_v, sem0)

                    def pair_k(j, _):
                        c0 = 2 * j
                        pltpu.async_copy(table_h.at[src_v.at[c0 + 1]],
                                         rows1_v, sem1)
                        pltpu.make_async_copy(table_h.at[src_v.at[c0]],
                                              rows0_v, sem0).wait()
                        pltpu.sync_copy(rows0_v, acc_sh.at[dst_v.at[c0]],
                                        add=True)

                        @pl.when(j < STAGE // 2 - 1)
                        def _():
                            pltpu.async_copy(table_h.at[src_v.at[c0 + 2]],
                                             rows0_v, sem0)
                        pltpu.make_async_copy(table_h.at[src_v.at[c0 + 1]],
                                              rows1_v, sem1).wait()
                        pltpu.sync_copy(rows1_v, acc_sh.at[dst_v.at[c0 + 1]],
                                        add=True)
                        return _
                    lax.fori_loop(0, STAGE // 2, pair_k, None)
                return _
            lax.fori_loop(0, NSUPER_PER_TILE, super_k, None)
            plsc.subcore_barrier()

            def wb_k(k, _):
                idx = s + k * NTILES

                @pl.when(idx < NINIT)
                def _():
                    r0 = pl.multiple_of(idx * INIT_CHUNK, 8)
                    pltpu.sync_copy(acc_sh.at[pl.ds(r0, INIT_CHUNK), :],
                                    bounce_v)
                    pltpu.sync_copy(bounce_v,
                                    out_h.at[pl.ds(r0, INIT_CHUNK), :])
                return _
            lax.fori_loop(0, NINIT_PER_TILE, wb_k, None)

        @pl.when(c == 0)
        def _():
            run(m0a_h, t0a_h, outa_h)

        @pl.when(c == 1)
        def _():
            run(m0b_h, t0b_h, outb_h)

    return sc_fn(edges3, m0a, m0b, t0a, t0b)


def _head_body(aa_ref, ab_ref, Wa_ref, Wb_ref, bo_ref, out_ref):
    tile = (
        jnp.dot(jnp.maximum(aa_ref[...], 0.0), Wa_ref[...],
                preferred_element_type=jnp.float32)
        + jnp.dot(jnp.maximum(ab_ref[...], 0.0), Wb_ref[...],
                  preferred_element_type=jnp.float32)
        + bo_ref[...])
    for k in range(8):
        out_ref[PACK * k:PACK * (k + 1), :] = tile[:, 2 * k:2 * k + 2]


def _head(acca8, accb8, Wa_blk, Wb_blk, bo_tile):
    full = lambda shape: pl.BlockSpec(shape, lambda i: (0,) * len(shape))
    return pl.pallas_call(
        _head_body,
        grid=(GRID,),
        in_specs=[
            pl.BlockSpec((PACK, 128), lambda i: (i, 0)),
            pl.BlockSpec((PACK, 128), lambda i: (i, 0)),
            full((128, 16)), full((128, 16)), full((1, 16)),
        ],
        out_specs=pl.BlockSpec((ROWS, 2), lambda i: (i, 0)),
        out_shape=jax.ShapeDtypeStruct((N, 2), jnp.float32),
    )(acca8, accb8, Wa_blk, Wb_blk, bo_tile)


def kernel(x_local, x_global, edge_index, batch, W_local, b_local, W_global,
           b_global, W_mix, b_mix, W_msg, b_msg, W_self, b_self, W_out, b_out):
    batch2 = batch.reshape(1, N)
    ge = ((edge_index & ~1023) | ((edge_index & 127) << 3)
          | ((edge_index >> 7) & 7))
    edges3 = ge.reshape(2, CHUNK_ROWS, CHUNK)
    m0a8, m0b8, t0a8, t0b8 = _encoder(
        x_local, batch2, x_global, W_local, b_local.reshape(1, HIDDEN),
        W_global, b_global.reshape(1, HIDDEN), W_mix, b_mix.reshape(1, HIDDEN),
        W_msg, b_msg.reshape(1, HIDDEN), W_self, b_self.reshape(1, HIDDEN))
    acca, accb = _sc_scatter(edges3,
                             m0a8.reshape(NPAD, HALF), m0b8.reshape(NPAD, HALF),
                             t0a8.reshape(NPAD, HALF), t0b8.reshape(NPAD, HALF))
    Wa_blk = jax.scipy.linalg.block_diag(*([W_out[:HALF]] * 8))
    Wb_blk = jax.scipy.linalg.block_diag(*([W_out[HALF:]] * 8))
    return _head(acca.reshape(N8P, 128), accb.reshape(N8P, 128),
                 Wa_blk, Wb_blk, jnp.tile(b_out, 8).reshape(1, 16))


# speedup vs baseline: 11.2317x; 1.1478x over previous
import functools

import jax
import jax.numpy as jnp
from jax import lax
from jax.experimental import pallas as pl
from jax.experimental.pallas import tpu as pltpu
from jax.experimental.pallas import tpu_sc as plsc

N = 100000
E = 1600000
B = 128
HIDDEN = 32
HALF = 16

ROWS = 1024
GRID = -(-N // ROWS)
PACK = ROWS // 8
NPAD = GRID * ROWS
N8P = GRID * PACK


NTILES = 16
CHUNK = 80
CHUNK_ROWS = E // CHUNK
STAGE = 40
NSUPER = CHUNK_ROWS // STAGE
NSUPER_PER_TILE = -(-NSUPER // NTILES)
INIT_CHUNK = 1024
NINIT = NPAD // INIT_CHUNK
NINIT_PER_TILE = -(-NINIT // NTILES)


def _encoder_body(x_ref, b_ref, xg_ref, Wl_ref, bl_ref, Wg_ref, bg_ref,
                  Wm_ref, bm_ref, Wmsg_ref, bmsg_ref, Ws_ref, bs_ref,
                  m0a_ref, m0b_ref, t0a_ref, t0b_ref):
    hg_tab = jnp.maximum(
        jnp.dot(xg_ref[...], Wg_ref[...], preferred_element_type=jnp.float32)
        + bg_ref[...], 0.0)
    Wm = Wm_ref[...]
    hl = jnp.maximum(
        jnp.dot(x_ref[...], Wl_ref[...], preferred_element_type=jnp.float32)
        + bl_ref[...], 0.0)
    b = b_ref[0, :]
    onehot = (b[:, None] == lax.broadcasted_iota(jnp.int32, (ROWS, B), 1)
              ).astype(jnp.float32)
    hgl = jnp.dot(onehot, hg_tab, preferred_element_type=jnp.float32)
    hi = hl * hgl
    h0 = jnp.maximum(
        jnp.dot(hl, Wm[0:32, :], preferred_element_type=jnp.float32)
        + jnp.dot(hgl, Wm[32:64, :], preferred_element_type=jnp.float32)
        + jnp.dot(hi, Wm[64:96, :], preferred_element_type=jnp.float32)
        + bm_ref[...], 0.0)
    m0 = jnp.maximum(
        jnp.dot(h0, Wmsg_ref[...], preferred_element_type=jnp.float32)
        + bmsg_ref[...], 0.0)
    t0 = m0 + jnp.dot(h0, Ws_ref[...], preferred_element_type=jnp.float32) \
        + bs_ref[...]
    valid = (pl.program_id(0) * ROWS
             + lax.broadcasted_iota(jnp.int32, (ROWS, 1), 0)) < N
    m0 = jnp.where(valid, m0, 0.0)
    t0 = jnp.where(valid, t0, 0.0)
    for k in range(8):
        m0k = m0[PACK * k:PACK * (k + 1), :]
        t0k = t0[PACK * k:PACK * (k + 1), :]
        m0a_ref[:, HALF * k:HALF * (k + 1)] = m0k[:, 0:HALF]
        m0b_ref[:, HALF * k:HALF * (k + 1)] = m0k[:, HALF:HIDDEN]
        t0a_ref[:, HALF * k:HALF * (k + 1)] = t0k[:, 0:HALF]
        t0b_ref[:, HALF * k:HALF * (k + 1)] = t0k[:, HALF:HIDDEN]


def _encoder(x_local, batch3, x_global, W_local, b_local, W_global, b_global,
             W_mix, b_mix, W_msg, b_msg, W_self, b_self):
    full = lambda shape: pl.BlockSpec(shape, lambda i: (0,) * len(shape))
    out = jax.ShapeDtypeStruct((N8P, 128), jnp.float32)
    return pl.pallas_call(
        _encoder_body,
        grid=(GRID,),
        in_specs=[
            pl.BlockSpec((ROWS, 128), lambda i: (i, 0)),
            pl.BlockSpec((1, ROWS), lambda i: (0, i)),
            full((B, 64)), full((128, HIDDEN)), full((1, HIDDEN)),
            full((64, HIDDEN)), full((1, HIDDEN)),
            full((96, HIDDEN)), full((1, HIDDEN)),
            full((HIDDEN, HIDDEN)), full((1, HIDDEN)),
            full((HIDDEN, HIDDEN)), full((1, HIDDEN)),
        ],
        out_specs=[pl.BlockSpec((PACK, 128), lambda i: (i, 0))] * 4,
        out_shape=[out, out, out, out],
    )(x_local, batch3, x_global, W_local, b_local, W_global, b_global,
      W_mix, b_mix, W_msg, b_msg, W_self, b_self)


def _sc_scatter(edges3, m0a, m0b, t0a, t0b):
    mesh = plsc.VectorSubcoreMesh(core_axis_name="c", subcore_axis_name="s")

    @functools.partial(
        pl.kernel, mesh=mesh,
        compiler_params=pltpu.CompilerParams(use_tc_tiling_on_sc=False),
        out_type=[jax.ShapeDtypeStruct((NPAD, HALF), jnp.float32),
                  jax.ShapeDtypeStruct((NPAD, HALF), jnp.float32)],
        scratch_types=[
            pltpu.VMEM((STAGE, CHUNK), jnp.int32),
            pltpu.VMEM((STAGE, CHUNK), jnp.int32),
            pltpu.VMEM((CHUNK, HALF), jnp.float32),
            pltpu.VMEM((CHUNK, HALF), jnp.float32),
            pltpu.VMEM((INIT_CHUNK, HALF), jnp.float32),
            pltpu.VMEM_SHARED((NPAD, HALF), jnp.float32),
            pltpu.SemaphoreType.DMA,
            pltpu.SemaphoreType.DMA,
        ],
    )
    def sc_fn(e3, m0a_h, m0b_h, t0a_h, t0b_h, outa_h, outb_h,
              src_v, dst_v, rows0_v, rows1_v, bounce_v, acc_sh, sem0, sem1):
        c = lax.axis_index("c")
        s = lax.axis_index("s")

        def run(table_h, t0_h, out_h):
            def init_k(k, _):
                idx = s + k * NTILES

                @pl.when(idx < NINIT)
                def _():
                    r0 = pl.multiple_of(idx * INIT_CHUNK, 8)
                    pltpu.sync_copy(t0_h.at[pl.ds(r0, INIT_CHUNK), :],
                                    bounce_v)
                    pltpu.sync_copy(bounce_v,
                                    acc_sh.at[pl.ds(r0, INIT_CHUNK), :])
                return _
            lax.fori_loop(0, NINIT_PER_TILE, init_k, None)
            plsc.subcore_barrier()

            def super_k(g, _):
                idx = s + g * NTILES

                @pl.when(idx < NSUPER)
                def _():
                    row0 = pl.multiple_of(idx * STAGE, 8)
                    pltpu.sync_copy(e3.at[0, pl.ds(row0, STAGE), :], src_v)
                    pltpu.sync_copy(e3.at[1, pl.ds(row0, STAGE), :], dst_v)

                    pltpu.async_copy(table_h.at[src_v.at[0]], rows0_v, sem0)

                    def pair_k(j, _):
                        c0 = 2 * j
                        pltpu.async_copy(table_h.at[src_v.at[c0 + 1]],
                                         rows1_v, sem1)
                        pltpu.make_async_copy(table_h.at[src_v.at[c0]],
                                              rows0_v, sem0).wait()
                        pltpu.sync_copy(rows0_v, acc_sh.at[dst_v.at[c0]],
                                        add=True)

                        @pl.when(j < STAGE // 2 - 1)
                        def _():
                            pltpu.async_copy(table_h.at[src_v.at[c0 + 2]],
                                             rows0_v, sem0)
                        pltpu.make_async_copy(table_h.at[src_v.at[c0 + 1]],
                                              rows1_v, sem1).wait()
                        pltpu.sync_copy(rows1_v, acc_sh.at[dst_v.at[c0 + 1]],
                                        add=True)
                        return _
                    lax.fori_loop(0, STAGE // 2, pair_k, None)
                return _
            lax.fori_loop(0, NSUPER_PER_TILE, super_k, None)
            plsc.subcore_barrier()

            def wb_k(k, _):
                idx = s + k * NTILES

                @pl.when(idx < NINIT)
                def _():
                    r0 = pl.multiple_of(idx * INIT_CHUNK, 8)
                    pltpu.sync_copy(acc_sh.at[pl.ds(r0, INIT_CHUNK), :],
                                    bounce_v)
                    pltpu.sync_copy(bounce_v,
                                    out_h.at[pl.ds(r0, INIT_CHUNK), :])
                return _
            lax.fori_loop(0, NINIT_PER_TILE, wb_k, None)

        @pl.when(c == 0)
        def _():
            run(m0a_h, t0a_h, outa_h)

        @pl.when(c == 1)
        def _():
            run(m0b_h, t0b_h, outb_h)

    return sc_fn(edges3, m0a, m0b, t0a, t0b)


def _head_body(aa_ref, ab_ref, Wa_ref, Wb_ref, bo_ref, out_ref):
    tile = (
        jnp.dot(jnp.maximum(aa_ref[...], 0.0), Wa_ref[...],
                preferred_element_type=jnp.float32)
        + jnp.dot(jnp.maximum(ab_ref[...], 0.0), Wb_ref[...],
                  preferred_element_type=jnp.float32)
        + bo_ref[...])
    for k in range(8):
        out_ref[PACK * k:PACK * (k + 1), :] = tile[:, 2 * k:2 * k + 2]


def _head(acca8, accb8, Wa_blk, Wb_blk, bo_tile):
    full = lambda shape: pl.BlockSpec(shape, lambda i: (0,) * len(shape))
    return pl.pallas_call(
        _head_body,
        grid=(GRID,),
        in_specs=[
            pl.BlockSpec((PACK, 128), lambda i: (i, 0)),
            pl.BlockSpec((PACK, 128), lambda i: (i, 0)),
            full((128, 16)), full((128, 16)), full((1, 16)),
        ],
        out_specs=pl.BlockSpec((ROWS, 2), lambda i: (i, 0)),
        out_shape=jax.ShapeDtypeStruct((N, 2), jnp.float32),
    )(acca8, accb8, Wa_blk, Wb_blk, bo_tile)


def kernel(x_local, x_global, edge_index, batch, W_local, b_local, W_global,
           b_global, W_mix, b_mix, W_msg, b_msg, W_self, b_self, W_out, b_out):
    batch2 = batch.reshape(1, N)
    ge = ((edge_index & ~1023) | ((edge_index & 127) << 3)
          | ((edge_index >> 7) & 7))
    edges3 = ge.reshape(2, CHUNK_ROWS, CHUNK)
    m0a8, m0b8, t0a8, t0b8 = _encoder(
        x_local, batch2, x_global, W_local, b_local.reshape(1, HIDDEN),
        W_global, b_global.reshape(1, HIDDEN), W_mix, b_mix.reshape(1, HIDDEN),
        W_msg, b_msg.reshape(1, HIDDEN), W_self, b_self.reshape(1, HIDDEN))
    acca, accb = _sc_scatter(edges3,
                             m0a8.reshape(NPAD, HALF), m0b8.reshape(NPAD, HALF),
                             t0a8.reshape(NPAD, HALF), t0b8.reshape(NPAD, HALF))
    Wa_blk = jax.scipy.linalg.block_diag(*([W_out[:HALF]] * 8))
    Wb_blk = jax.scipy.linalg.block_diag(*([W_out[HALF:]] * 8))
    return _head(acca.reshape(N8P, 128), accb.reshape(N8P, 128),
                 Wa_blk, Wb_blk, jnp.tile(b_out, 8).reshape(1, 16))


# speedup vs baseline: 16.0788x; 1.4316x over previous
import functools

import jax
import jax.numpy as jnp
from jax import lax
from jax.experimental import pallas as pl
from jax.experimental.pallas import tpu as pltpu
from jax.experimental.pallas import tpu_sc as plsc

N = 100000
E = 1600000
B = 128
HIDDEN = 32
HALF = 16

ROWS = 1024
GRID = -(-N // ROWS)
PACK = ROWS // 8
NPAD = GRID * ROWS
N8P = GRID * PACK


NTILES = 16
CHUNK = 80
CHUNK_ROWS = E // CHUNK
STAGE = 40
RING = 8
LOOK = 4
NSUPER = CHUNK_ROWS // STAGE
NSUPER_PER_TILE = -(-NSUPER // NTILES)
INIT_CHUNK = 512
NINIT = NPAD // INIT_CHUNK
NINIT_PER_TILE = -(-NINIT // NTILES)


def _encoder_body(x_ref, b_ref, xg_ref, Wl_ref, bl_ref, Wg_ref, bg_ref,
                  Wm_ref, bm_ref, Wmsg_ref, bmsg_ref, Ws_ref, bs_ref,
                  m0a_ref, m0b_ref, t0a_ref, t0b_ref):
    hg_tab = jnp.maximum(
        jnp.dot(xg_ref[...], Wg_ref[...], preferred_element_type=jnp.float32)
        + bg_ref[...], 0.0)
    Wm = Wm_ref[...]
    hl = jnp.maximum(
        jnp.dot(x_ref[...], Wl_ref[...], preferred_element_type=jnp.float32)
        + bl_ref[...], 0.0)
    b = b_ref[0, :]
    onehot = (b[:, None] == lax.broadcasted_iota(jnp.int32, (ROWS, B), 1)
              ).astype(jnp.float32)
    hgl = jnp.dot(onehot, hg_tab, preferred_element_type=jnp.float32)
    hi = hl * hgl
    h0 = jnp.maximum(
        jnp.dot(hl, Wm[0:32, :], preferred_element_type=jnp.float32)
        + jnp.dot(hgl, Wm[32:64, :], preferred_element_type=jnp.float32)
        + jnp.dot(hi, Wm[64:96, :], preferred_element_type=jnp.float32)
        + bm_ref[...], 0.0)
    m0 = jnp.maximum(
        jnp.dot(h0, Wmsg_ref[...], preferred_element_type=jnp.float32)
        + bmsg_ref[...], 0.0)
    t0 = m0 + jnp.dot(h0, Ws_ref[...], preferred_element_type=jnp.float32) \
        + bs_ref[...]
    valid = (pl.program_id(0) * ROWS
             + lax.broadcasted_iota(jnp.int32, (ROWS, 1), 0)) < N
    m0 = jnp.where(valid, m0, 0.0)
    t0 = jnp.where(valid, t0, 0.0)
    for k in range(8):
        m0k = m0[PACK * k:PACK * (k + 1), :]
        t0k = t0[PACK * k:PACK * (k + 1), :]
        m0a_ref[:, HALF * k:HALF * (k + 1)] = m0k[:, 0:HALF]
        m0b_ref[:, HALF * k:HALF * (k + 1)] = m0k[:, HALF:HIDDEN]
        t0a_ref[:, HALF * k:HALF * (k + 1)] = t0k[:, 0:HALF]
        t0b_ref[:, HALF * k:HALF * (k + 1)] = t0k[:, HALF:HIDDEN]


def _encoder(x_local, batch3, x_global, W_local, b_local, W_global, b_global,
             W_mix, b_mix, W_msg, b_msg, W_self, b_self):
    full = lambda shape: pl.BlockSpec(shape, lambda i: (0,) * len(shape))
    out = jax.ShapeDtypeStruct((N8P, 128), jnp.float32)
    return pl.pallas_call(
        _encoder_body,
        grid=(GRID,),
        in_specs=[
            pl.BlockSpec((ROWS, 128), lambda i: (i, 0)),
            pl.BlockSpec((1, ROWS), lambda i: (0, i)),
            full((B, 64)), full((128, HIDDEN)), full((1, HIDDEN)),
            full((64, HIDDEN)), full((1, HIDDEN)),
            full((96, HIDDEN)), full((1, HIDDEN)),
            full((HIDDEN, HIDDEN)), full((1, HIDDEN)),
            full((HIDDEN, HIDDEN)), full((1, HIDDEN)),
        ],
        out_specs=[pl.BlockSpec((PACK, 128), lambda i: (i, 0))] * 4,
        out_shape=[out, out, out, out],
    )(x_local, batch3, x_global, W_local, b_local, W_global, b_global,
      W_mix, b_mix, W_msg, b_msg, W_self, b_self)


def _sc_scatter(edges3, m0a, m0b, t0a, t0b):
    mesh = plsc.VectorSubcoreMesh(core_axis_name="c", subcore_axis_name="s")

    @functools.partial(
        pl.kernel, mesh=mesh,
        compiler_params=pltpu.CompilerParams(use_tc_tiling_on_sc=False),
        out_type=[jax.ShapeDtypeStruct((NPAD, HALF), jnp.float32),
                  jax.ShapeDtypeStruct((NPAD, HALF), jnp.float32)],
        scratch_types=[
            pltpu.VMEM((STAGE, CHUNK), jnp.int32),
            pltpu.VMEM((STAGE, CHUNK), jnp.int32),
            pltpu.VMEM((RING, CHUNK, HALF), jnp.float32),
            pltpu.VMEM((INIT_CHUNK, HALF), jnp.float32),
            pltpu.VMEM_SHARED((NPAD, HALF), jnp.float32),
            pltpu.SemaphoreType.DMA((RING,)),
            pltpu.SemaphoreType.DMA((RING,)),
        ],
    )
    def sc_fn(e3, m0a_h, m0b_h, t0a_h, t0b_h, outa_h, outb_h,
              src_v, dst_v, rows_v, bounce_v, acc_sh, sem_g, sem_s):
        c = lax.axis_index("c")
        s = lax.axis_index("s")

        def run(table_h, t0_h, out_h):
            def init_k(k, _):
                idx = s + k * NTILES

                @pl.when(idx < NINIT)
                def _():
                    r0 = pl.multiple_of(idx * INIT_CHUNK, 8)
                    pltpu.sync_copy(t0_h.at[pl.ds(r0, INIT_CHUNK), :],
                                    bounce_v)
                    pltpu.sync_copy(bounce_v,
                                    acc_sh.at[pl.ds(r0, INIT_CHUNK), :])
                return _
            lax.fori_loop(0, NINIT_PER_TILE, init_k, None)
            plsc.subcore_barrier()

            def super_k(g, _):
                idx = s + g * NTILES

                @pl.when(idx < NSUPER)
                def _():
                    row0 = pl.multiple_of(idx * STAGE, 8)
                    pltpu.sync_copy(e3.at[0, pl.ds(row0, STAGE), :], src_v)
                    pltpu.sync_copy(e3.at[1, pl.ds(row0, STAGE), :], dst_v)

                    for c in range(LOOK):
                        pltpu.async_copy(table_h.at[src_v.at[c]],
                                         rows_v.at[c], sem_g.at[c])

                    def chunk_k(j, _):
                        bslot = j % RING
                        pltpu.make_async_copy(table_h.at[src_v.at[j]],
                                              rows_v.at[bslot],
                                              sem_g.at[bslot]).wait()
                        pltpu.async_copy(rows_v.at[bslot],
                                         acc_sh.at[dst_v.at[j]],
                                         sem_s.at[bslot], add=True)

                        @pl.when(j + LOOK < STAGE)
                        def _():
                            nslot = (j + LOOK) % RING
                            @pl.when(j >= LOOK)
                            def _():
                                pltpu.make_async_copy(
                                    rows_v.at[nslot],
                                    acc_sh.at[dst_v.at[j - LOOK]],
                                    sem_s.at[nslot]).wait()
                            pltpu.async_copy(table_h.at[src_v.at[j + LOOK]],
                                             rows_v.at[nslot],
                                             sem_g.at[nslot])
                        return _
                    lax.fori_loop(0, STAGE, chunk_k, None)
                    for c in range(STAGE - 2 * LOOK, STAGE):
                        pltpu.make_async_copy(rows_v.at[c % RING],
                                              acc_sh.at[dst_v.at[c]],
                                              sem_s.at[c % RING]).wait()
                return _
            lax.fori_loop(0, NSUPER_PER_TILE, super_k, None)
            plsc.subcore_barrier()

            def wb_k(k, _):
                idx = s + k * NTILES

                @pl.when(idx < NINIT)
                def _():
                    r0 = pl.multiple_of(idx * INIT_CHUNK, 8)
                    pltpu.sync_copy(acc_sh.at[pl.ds(r0, INIT_CHUNK), :],
                                    bounce_v)
                    pltpu.sync_copy(bounce_v,
                                    out_h.at[pl.ds(r0, INIT_CHUNK), :])
                return _
            lax.fori_loop(0, NINIT_PER_TILE, wb_k, None)

        @pl.when(c == 0)
        def _():
            run(m0a_h, t0a_h, outa_h)

        @pl.when(c == 1)
        def _():
            run(m0b_h, t0b_h, outb_h)

    return sc_fn(edges3, m0a, m0b, t0a, t0b)


def _head_body(aa_ref, ab_ref, Wa_ref, Wb_ref, bo_ref, out_ref):
    tile = (
        jnp.dot(jnp.maximum(aa_ref[...], 0.0), Wa_ref[...],
                preferred_element_type=jnp.float32)
        + jnp.dot(jnp.maximum(ab_ref[...], 0.0), Wb_ref[...],
                  preferred_element_type=jnp.float32)
        + bo_ref[...])
    for k in range(8):
        out_ref[PACK * k:PACK * (k + 1), :] = tile[:, 2 * k:2 * k + 2]


def _head(acca8, accb8, Wa_blk, Wb_blk, bo_tile):
    full = lambda shape: pl.BlockSpec(shape, lambda i: (0,) * len(shape))
    return pl.pallas_call(
        _head_body,
        grid=(GRID,),
        in_specs=[
            pl.BlockSpec((PACK, 128), lambda i: (i, 0)),
            pl.BlockSpec((PACK, 128), lambda i: (i, 0)),
            full((128, 16)), full((128, 16)), full((1, 16)),
        ],
        out_specs=pl.BlockSpec((ROWS, 2), lambda i: (i, 0)),
        out_shape=jax.ShapeDtypeStruct((N, 2), jnp.float32),
    )(acca8, accb8, Wa_blk, Wb_blk, bo_tile)


def kernel(x_local, x_global, edge_index, batch, W_local, b_local, W_global,
           b_global, W_mix, b_mix, W_msg, b_msg, W_self, b_self, W_out, b_out):
    batch2 = batch.reshape(1, N)
    ge = ((edge_index & ~1023) | ((edge_index & 127) << 3)
          | ((edge_index >> 7) & 7))
    edges3 = ge.reshape(2, CHUNK_ROWS, CHUNK)
    m0a8, m0b8, t0a8, t0b8 = _encoder(
        x_local, batch2, x_global, W_local, b_local.reshape(1, HIDDEN),
        W_global, b_global.reshape(1, HIDDEN), W_mix, b_mix.reshape(1, HIDDEN),
        W_msg, b_msg.reshape(1, HIDDEN), W_self, b_self.reshape(1, HIDDEN))
    acca, accb = _sc_scatter(edges3,
                             m0a8.reshape(NPAD, HALF), m0b8.reshape(NPAD, HALF),
                             t0a8.reshape(NPAD, HALF), t0b8.reshape(NPAD, HALF))
    Wa_blk = jax.scipy.linalg.block_diag(*([W_out[:HALF]] * 8))
    Wb_blk = jax.scipy.linalg.block_diag(*([W_out[HALF:]] * 8))
    return _head(acca.reshape(N8P, 128), accb.reshape(N8P, 128),
                 Wa_blk, Wb_blk, jnp.tile(b_out, 8).reshape(1, 16))
